# baseline (speedup 1.0000x reference)
import jax
import jax.numpy as jnp
from jax.experimental import pallas as pl
from jax.experimental.pallas import tpu as pltpu

_SQRT_HALF = 0.7071067811865476
_NB = 2


def _mlp_body(x_bf, w1_ref, b1_ref, w2_ref, b2_ref):
    h = jnp.dot(w1_ref[...], x_bf, preferred_element_type=jnp.float32)
    h = h + b1_ref[...]
    g = 0.5 * h * (1.0 + jax.lax.erf(h * _SQRT_HALF))
    y = jnp.dot(w2_ref[...], g.astype(jnp.bfloat16),
                preferred_element_type=jnp.float32)
    return (y + b2_ref[...]).astype(jnp.bfloat16)


def _pipeline_kernel(x_hbm, w1_ref, b1_ref, w2_ref, b2_ref, o_hbm,
                     in_buf, out_buf, in_sem, out_sem):
    n_steps = x_hbm.shape[0] // _NB

    def copy_in(s, slot):
        return pltpu.make_async_copy(
            x_hbm.at[pl.ds(s * _NB, _NB)], in_buf.at[slot], in_sem.at[slot])

    def copy_out(s, slot):
        return pltpu.make_async_copy(
            out_buf.at[slot], o_hbm.at[pl.ds(s * _NB, _NB)], out_sem.at[slot])

    copy_in(0, 0).start()

    def step(s, _):
        slot = jax.lax.rem(s, 2)
        nslot = jax.lax.rem(s + 1, 2)

        @pl.when(s + 1 < n_steps)
        def _():
            copy_in(s + 1, nslot).start()

        copy_in(s, slot).wait()

        @pl.when(s >= 2)
        def _():
            copy_out(s - 2, slot).wait()

        for i in range(_NB):
            out_buf[slot, i] = _mlp_body(in_buf[slot, i],
                                         w1_ref, b1_ref, w2_ref, b2_ref)

        copy_out(s, slot).start()
        return 0

    jax.lax.fori_loop(0, n_steps, step, 0)
    copy_out(n_steps - 2, jax.lax.rem(n_steps - 2, 2)).wait()
    copy_out(n_steps - 1, jax.lax.rem(n_steps - 1, 2)).wait()


def kernel(x, w1, b1, w2, b2):
    B, Cin, H, W = x.shape
    hidden = w1.shape[0]
    Cout = w2.shape[0]
    HW = H * W

    x3 = x.reshape(B, Cin, HW).astype(jnp.bfloat16)

    vmem_full = pl.BlockSpec(memory_space=pltpu.MemorySpace.VMEM)
    flops = 2 * B * HW * (Cin * hidden + hidden * Cout)
    cost = pl.CostEstimate(flops=flops,
                           transcendentals=B * HW * hidden,
                           bytes_accessed=2 * B * HW * (Cin + Cout))

    out3 = pl.pallas_call(
        _pipeline_kernel,
        out_shape=jax.ShapeDtypeStruct((B, Cout, HW), jnp.bfloat16),
        in_specs=[
            pl.BlockSpec(memory_space=pltpu.MemorySpace.HBM),
            vmem_full, vmem_full, vmem_full, vmem_full,
        ],
        out_specs=pl.BlockSpec(memory_space=pltpu.MemorySpace.HBM),
        scratch_shapes=[
            pltpu.VMEM((2, _NB, Cin, HW), jnp.bfloat16),
            pltpu.VMEM((2, _NB, Cout, HW), jnp.bfloat16),
            pltpu.SemaphoreType.DMA((2,)),
            pltpu.SemaphoreType.DMA((2,)),
        ],
        cost_estimate=cost,
    )(x3, w1.astype(jnp.bfloat16), b1, w2.astype(jnp.bfloat16), b2)

    return out3.astype(jnp.float32).reshape(B, Cout, H, W)


# speedup vs baseline: 1.1031x; 1.1031x over previous
import jax
import jax.numpy as jnp
from jax.experimental import pallas as pl
from jax.experimental.pallas import tpu as pltpu

_SQRT_HALF = 0.7071067811865476
_NB = 2


def _mlp_kernel(x_ref, w1_ref, b1_ref, w2_ref, b2_ref, o_ref):
    for i in range(_NB):
        x = x_ref[i]
        h = jnp.dot(w1_ref[...], x, preferred_element_type=jnp.float32)
        h = h + b1_ref[...]
        g = 0.5 * h * (1.0 + jax.lax.erf(h * _SQRT_HALF))
        y = jnp.dot(w2_ref[...], g, preferred_element_type=jnp.float32)
        o_ref[i] = (y + b2_ref[...]).astype(jnp.bfloat16)


def kernel(x, w1, b1, w2, b2):
    B, Cin, H, W = x.shape
    hidden = w1.shape[0]
    Cout = w2.shape[0]
    HW = H * W

    x3 = x.reshape(B, Cin, HW)

    full2d = lambda shape: pl.BlockSpec(shape, lambda b: (0, 0))
    flops = 2 * B * HW * (Cin * hidden + hidden * Cout)
    cost = pl.CostEstimate(flops=flops,
                           transcendentals=B * HW * hidden,
                           bytes_accessed=4 * B * HW * Cin + 2 * B * HW * Cout)

    out3 = pl.pallas_call(
        _mlp_kernel,
        out_shape=jax.ShapeDtypeStruct((B, Cout, HW), jnp.bfloat16),
        grid=(B // _NB,),
        in_specs=[
            pl.BlockSpec((_NB, Cin, HW), lambda b: (b, 0, 0)),
            full2d((hidden, Cin)),
            full2d((hidden, 1)),
            full2d((Cout, hidden)),
            full2d((Cout, 1)),
        ],
        out_specs=pl.BlockSpec((_NB, Cout, HW), lambda b: (b, 0, 0)),
        compiler_params=pltpu.CompilerParams(
            dimension_semantics=("parallel",),
        ),
        cost_estimate=cost,
    )(x3, w1, b1, w2, b2)

    return out3.astype(jnp.float32).reshape(B, Cout, H, W)
